# baseline (device time: 29946 ns/iter reference)
import jax
import jax.numpy as jnp
from jax import lax
from jax.experimental import pallas as pl
from jax.experimental.pallas import tpu as pltpu

N_DEV = 16
N_TOK = 2048
D = 512
H = 1024
N_EXP = 64
E_LOCAL = N_EXP // N_DEV
CAP = 25
CAP_PAD = 32
BLOCK = E_LOCAL * CAP_PAD
TB = 128
N_TB = N_TOK // TB
TPB = N_TOK // N_DEV
PAD = 32


def _moe_kernel(x, route_idx, expert_W):
    def body(x_ref, e_ref, w_ref, out_ref, dst_ref, a2a_ref,
             send_sems, recv_sems, loc_sem):
        my = lax.axis_index("i")

        barrier_sem = pltpu.get_barrier_semaphore()
        for o in range(1, N_DEV):
            pl.semaphore_signal(
                barrier_sem, inc=1,
                device_id=(lax.rem(my + o, N_DEV),),
                device_id_type=pl.DeviceIdType.MESH,
            )
        pl.semaphore_wait(barrier_sem, N_DEV - 1)

        eid = jnp.arange(N_EXP, dtype=jnp.int32)[None, :]
        tri = lax.broadcasted_iota(jnp.int32, (TB, TB), 1) <= \
            lax.broadcasted_iota(jnp.int32, (TB, TB), 0)
        l_incl = tri.astype(jnp.bfloat16)
        stri = lax.broadcasted_iota(jnp.int32, (TB, TB), 1) < \
            lax.broadcasted_iota(jnp.int32, (TB, TB), 0)
        l_strict = stri.astype(jnp.bfloat16)
        tri16 = lax.broadcasted_iota(jnp.int32, (N_TB, N_TB), 1) < \
            lax.broadcasted_iota(jnp.int32, (N_TB, N_TB), 0)
        l_excl16 = tri16.astype(jnp.bfloat16)
        rep = (lax.broadcasted_iota(jnp.int32, (N_TOK, N_TB), 0) // TB ==
               lax.broadcasted_iota(jnp.int32, (N_TOK, N_TB), 1))
        rep_bf = rep.astype(jnp.bfloat16)

        my_eid = my * E_LOCAL + jnp.arange(E_LOCAL, dtype=jnp.int32)[None, :]
        oh4 = (e_ref[:, :] == my_eid)
        oh4_bf = oh4.astype(jnp.bfloat16)
        oh4_32 = oh4.astype(jnp.float32)
        intra4 = []
        for b in range(N_TB):
            intra4.append(lax.dot_general(
                l_incl, oh4_bf[b * TB:(b + 1) * TB, :],
                (((1,), (0,)), ((), ())),
                preferred_element_type=jnp.float32,
            ))
        bs4 = jnp.concatenate([i4[TB - 1:TB, :] for i4 in intra4], axis=0)
        off4 = lax.dot_general(
            l_excl16, bs4.astype(jnp.bfloat16),
            (((1,), (0,)), ((), ())),
            preferred_element_type=jnp.float32,
        )
        off4_rep = lax.dot_general(
            rep_bf, off4.astype(jnp.bfloat16),
            (((1,), (0,)), ((), ())),
            preferred_element_type=jnp.float32,
        )
        intra_all = jnp.concatenate(intra4, axis=0)
        rank4 = jnp.sum(
            oh4_32 * (intra_all + off4_rep), axis=1, keepdims=True
        ) - 1.0
        rank4_i = rank4.astype(jnp.int32)
        kept4 = (rank4_i >= 0) & (rank4_i < CAP)

        e_all = e_ref[:, :]
        slot = (e_all % E_LOCAL) * CAP_PAD + rank4_i
        slot = jnp.where(kept4, slot, -1)
        sel_t = (slot == lax.broadcasted_iota(jnp.int32, (N_TOK, BLOCK), 1))
        sel_t = sel_t.astype(jnp.bfloat16)
        xg = lax.dot_general(
            sel_t, x_ref[:, :].astype(jnp.bfloat16),
            (((0,), (0,)), ((), ())),
            preferred_element_type=jnp.float32,
        ).astype(jnp.bfloat16)

        compact = jnp.concatenate([
            jnp.dot(
                xg[k * CAP_PAD:(k + 1) * CAP_PAD, :],
                w_ref[k].astype(jnp.bfloat16),
                preferred_element_type=jnp.float32,
            ).astype(jnp.bfloat16)
            for k in range(E_LOCAL)
        ], axis=0)

        oh4k_bf = (oh4_32 * kept4.astype(jnp.float32)).astype(jnp.bfloat16)
        cnt2 = lax.dot_general(
            rep_bf, oh4k_bf, (((0,), (0,)), ((), ())),
            preferred_element_type=jnp.float32,
        )
        u4 = (lax.broadcasted_iota(jnp.int32, (E_LOCAL, E_LOCAL), 0) <
              lax.broadcasted_iota(jnp.int32, (E_LOCAL, E_LOCAL), 1))
        cnt2_prior = lax.dot_general(
            cnt2.astype(jnp.bfloat16), u4.astype(jnp.bfloat16),
            (((1,), (0,)), ((), ())),
            preferred_element_type=jnp.float32,
        )
        prior_tok = jnp.sum(
            oh4_32 * lax.dot_general(
                rep_bf, cnt2_prior.astype(jnp.bfloat16),
                (((1,), (0,)), ((), ())),
                preferred_element_type=jnp.float32,
            ),
            axis=1, keepdims=True,
        )
        base = jnp.sum(
            oh4_32 * jnp.minimum(off4_rep, float(CAP)),
            axis=1, keepdims=True,
        )
        owner = lax.broadcasted_iota(jnp.int32, (N_TOK, 1), 0) // TPB
        pos = owner * PAD + prior_tok.astype(jnp.int32) + rank4_i - \
            base.astype(jnp.int32)
        pos = jnp.where(kept4, pos, -1)

        m_pos = (pos == lax.broadcasted_iota(
            jnp.int32, (N_TOK, N_DEV * PAD), 1)).astype(jnp.bfloat16)
        q_perm = lax.dot_general(
            m_pos, sel_t, (((0,), (0,)), ((), ())),
            preferred_element_type=jnp.float32,
        ).astype(jnp.bfloat16)
        dst_ref[:, :] = lax.dot_general(
            q_perm, compact, (((1,), (0,)), ((), ())),
            preferred_element_type=jnp.float32,
        ).astype(jnp.bfloat16)

        in_flight = []
        for o in range(1, N_DEV):
            dst_dev = lax.rem(my + o, N_DEV)
            d = pltpu.make_async_remote_copy(
                src_ref=dst_ref.at[pl.ds(dst_dev * PAD, PAD)],
                dst_ref=a2a_ref.at[o - 1],
                send_sem=send_sems.at[o - 1],
                recv_sem=recv_sems.at[o - 1],
                device_id=(dst_dev,),
                device_id_type=pl.DeviceIdType.MESH,
            )
            d.start()
            in_flight.append(d)
        loc = pltpu.make_async_copy(
            dst_ref.at[pl.ds(my * PAD, PAD)], a2a_ref.at[N_DEV - 1], loc_sem
        )
        loc.start()

        oh_all_bf = (e_ref[:, :] == eid).astype(jnp.bfloat16)
        bs_all = lax.dot_general(
            rep_bf, oh_all_bf, (((0,), (0,)), ((), ())),
            preferred_element_type=jnp.float32,
        )
        wrow = (lax.broadcasted_iota(jnp.int32, (1, N_TB), 1) < my)
        off_my = lax.dot_general(
            wrow.astype(jnp.bfloat16), bs_all.astype(jnp.bfloat16),
            (((1,), (0,)), ((), ())),
            preferred_element_type=jnp.float32,
        )
        e_my = e_ref[pl.ds(my * TPB, TPB), :]
        oh_my = (e_my == eid)
        oh_my_32 = oh_my.astype(jnp.float32)
        intra_my = lax.dot_general(
            l_incl, oh_my.astype(jnp.bfloat16),
            (((1,), (0,)), ((), ())),
            preferred_element_type=jnp.float32,
        )
        rank_my = jnp.sum(
            oh_my_32 * (intra_my + off_my), axis=1, keepdims=True
        ) - 1.0
        r_my = rank_my.astype(jnp.int32)
        keep_my = r_my < CAP

        ohk_my_bf = (oh_my_32 * keep_my.astype(jnp.float32)).astype(
            jnp.bfloat16
        )
        c_my = jnp.sum(ohk_my_bf.astype(jnp.float32), axis=0, keepdims=True)
        g64r = lax.broadcasted_iota(jnp.int32, (N_EXP, N_EXP), 0)
        g64c = lax.broadcasted_iota(jnp.int32, (N_EXP, N_EXP), 1)
        w64 = ((g64r // E_LOCAL == g64c // E_LOCAL) & (g64r < g64c))
        prior_my = lax.dot_general(
            c_my.astype(jnp.bfloat16), w64.astype(jnp.bfloat16),
            (((1,), (0,)), ((), ())),
            preferred_element_type=jnp.float32,
        )
        r2 = lax.dot_general(
            l_strict, ohk_my_bf, (((1,), (0,)), ((), ())),
            preferred_element_type=jnp.float32,
        )
        seg_row = jnp.sum(
            oh_my_32 * (prior_my + r2), axis=1, keepdims=True
        ).astype(jnp.int32)
        q_my = e_my // E_LOCAL
        o_my = lax.rem(my - q_my - 1 + N_DEV, N_DEV)
        flat = o_my * PAD + seg_row
        flat = jnp.where(keep_my, flat, -1)
        sel2 = (flat == lax.broadcasted_iota(
            jnp.int32, (TPB, N_DEV * PAD), 1)).astype(jnp.bfloat16)

        for o in range(1, N_DEV):
            pltpu.make_async_remote_copy(
                src_ref=dst_ref.at[pl.ds(my * PAD, PAD)],
                dst_ref=a2a_ref.at[o - 1],
                send_sem=send_sems.at[o - 1],
                recv_sem=recv_sems.at[o - 1],
                device_id=(lax.rem(my + o, N_DEV),),
                device_id_type=pl.DeviceIdType.MESH,
            ).wait_recv()
        loc.wait()

        g = a2a_ref[...].reshape(N_DEV * PAD, H)
        out_ref[:, :] = lax.dot_general(
            sel2, g, (((1,), (0,)), ((), ())),
            preferred_element_type=jnp.float32,
        )

        for d in in_flight:
            d.wait_send()

    return pl.pallas_call(
        body,
        out_shape=jax.ShapeDtypeStruct((TPB, H), jnp.float32),
        in_specs=[
            pl.BlockSpec(memory_space=pltpu.VMEM),
            pl.BlockSpec(memory_space=pltpu.VMEM),
            pl.BlockSpec(memory_space=pltpu.VMEM),
        ],
        out_specs=pl.BlockSpec(memory_space=pltpu.VMEM),
        scratch_shapes=[
            pltpu.VMEM((N_DEV * PAD, H), jnp.bfloat16),
            pltpu.VMEM((N_DEV, PAD, H), jnp.bfloat16),
            pltpu.SemaphoreType.DMA((N_DEV - 1,)),
            pltpu.SemaphoreType.DMA((N_DEV - 1,)),
            pltpu.SemaphoreType.DMA,
        ],
        compiler_params=pltpu.CompilerParams(collective_id=0),
    )(x, route_idx, expert_W)


def kernel(x, router_W, route_idx, expert_W):
    return _moe_kernel(x, route_idx.astype(jnp.int32), expert_W)
